# Initial kernel scaffold; baseline (speedup 1.0000x reference)
import jax
import jax.numpy as jnp
from jax.experimental import pallas as pl


def kernel(x, edge_index, W1, b1, W2, b2):
    raise NotImplementedError("write your pallas kernel here")


# speedup vs baseline: 81.0573x; 81.0573x over previous
import functools

import jax
import jax.numpy as jnp
from jax import lax
from jax.experimental import pallas as pl
from jax.experimental.pallas import tpu as pltpu
from jax.experimental.pallas import tpu_sc as plsc

N = 100_000
E = 3_200_000
NC, NS = 2, 16
NW = NC * NS
ROWS = 784
EPT = ROWS * 128
EPAD = NW * EPT
T = 100_352
CH = 8
TSL = T // NS


def _mesh():
    return plsc.VectorSubcoreMesh(core_axis_name="c", subcore_axis_name="s",
                                  num_cores=NC, num_subcores=NS)


def _sc_deg_body(dst_hbm, zeros_hbm, out_hbm, ones_v, idx_v, deg_sp, sem):
    cid = lax.axis_index("c")
    sid = lax.axis_index("s")
    wid = cid * NS + sid

    for c in range(0, 128, 16):
        ones_v[0, pl.ds(c, 16)] = jnp.full((16,), 1.0, jnp.float32)

    @pl.when(sid == 0)
    def _():
        pltpu.sync_copy(zeros_hbm, deg_sp)
    plsc.subcore_barrier()

    @pl.loop(0, ROWS, step=CH)
    def _(r0):
        pltpu.sync_copy(dst_hbm.at[wid, pl.ds(r0, CH)], idx_v)
        for j in range(CH):
            pltpu.sync_copy(ones_v.at[0], deg_sp.at[idx_v.at[j]], add=True)

    plsc.subcore_barrier()
    pltpu.sync_copy(deg_sp.at[pl.ds(sid * TSL, TSL)],
                    out_hbm.at[cid, pl.ds(sid * TSL, TSL)])


def _sc_agg1_body(src_hbm, dst_hbm, s_hbm, zeros_hbm, out_hbm,
                  sidx_v, didx_v, vals_v, s_sp, acc_sp, sem):
    cid = lax.axis_index("c")
    sid = lax.axis_index("s")
    wid = cid * NS + sid

    @pl.when(sid == 0)
    def _():
        pltpu.sync_copy(s_hbm, s_sp)
        pltpu.sync_copy(zeros_hbm, acc_sp)
    plsc.subcore_barrier()

    @pl.loop(0, ROWS, step=CH)
    def _(r0):
        pltpu.sync_copy(src_hbm.at[wid, pl.ds(r0, CH)], sidx_v)
        pltpu.sync_copy(dst_hbm.at[wid, pl.ds(r0, CH)], didx_v)
        for j in range(CH):
            pltpu.sync_copy(s_sp.at[sidx_v.at[j]], vals_v.at[j])
            pltpu.sync_copy(vals_v.at[j], acc_sp.at[didx_v.at[j]], add=True)

    plsc.subcore_barrier()
    pltpu.sync_copy(acc_sp.at[pl.ds(sid * TSL, TSL)],
                    out_hbm.at[cid, pl.ds(sid * TSL, TSL)])


def _sc_agg2_body(src_hbm, dst_hbm, p0_hbm, p1_hbm, zeros_hbm, out_hbm,
                  sidx_v, didx_v, v0_v, v1_v, p0_sp, p1_sp, a0_sp, a1_sp, sem):
    cid = lax.axis_index("c")
    sid = lax.axis_index("s")
    wid = cid * NS + sid

    @pl.when(sid == 0)
    def _():
        pltpu.sync_copy(p0_hbm, p0_sp)
        pltpu.sync_copy(p1_hbm, p1_sp)
        pltpu.sync_copy(zeros_hbm, a0_sp)
        pltpu.sync_copy(zeros_hbm, a1_sp)
    plsc.subcore_barrier()

    @pl.loop(0, ROWS, step=CH)
    def _(r0):
        pltpu.sync_copy(src_hbm.at[wid, pl.ds(r0, CH)], sidx_v)
        pltpu.sync_copy(dst_hbm.at[wid, pl.ds(r0, CH)], didx_v)
        for j in range(CH):
            pltpu.sync_copy(p0_sp.at[sidx_v.at[j]], v0_v.at[j])
            pltpu.sync_copy(p1_sp.at[sidx_v.at[j]], v1_v.at[j])
            pltpu.sync_copy(v0_v.at[j], a0_sp.at[didx_v.at[j]], add=True)
            pltpu.sync_copy(v1_v.at[j], a1_sp.at[didx_v.at[j]], add=True)

    plsc.subcore_barrier()
    pltpu.sync_copy(a0_sp.at[pl.ds(sid * TSL, TSL)],
                    out_hbm.at[cid, 0, pl.ds(sid * TSL, TSL)])
    pltpu.sync_copy(a1_sp.at[pl.ds(sid * TSL, TSL)],
                    out_hbm.at[cid, 1, pl.ds(sid * TSL, TSL)])


def _sc_deg(dstp, zeros_t):
    return pl.kernel(
        _sc_deg_body,
        out_type=jax.ShapeDtypeStruct((NC, T), jnp.float32),
        mesh=_mesh(),
        scratch_types=[
            pltpu.VMEM((1, 128), jnp.float32),
            pltpu.VMEM((CH, 128), jnp.int32),
            pltpu.VMEM_SHARED((T,), jnp.float32),
            pltpu.SemaphoreType.DMA,
        ],
    )(dstp, zeros_t)


def _sc_agg1(srcp, dstp, s_t, zeros_t):
    return pl.kernel(
        _sc_agg1_body,
        out_type=jax.ShapeDtypeStruct((NC, T), jnp.float32),
        mesh=_mesh(),
        scratch_types=[
            pltpu.VMEM((CH, 128), jnp.int32),
            pltpu.VMEM((CH, 128), jnp.int32),
            pltpu.VMEM((CH, 128), jnp.float32),
            pltpu.VMEM_SHARED((T,), jnp.float32),
            pltpu.VMEM_SHARED((T,), jnp.float32),
            pltpu.SemaphoreType.DMA,
        ],
    )(srcp, dstp, s_t, zeros_t)


def _sc_agg2(srcp, dstp, p0_t, p1_t, zeros_t):
    return pl.kernel(
        _sc_agg2_body,
        out_type=jax.ShapeDtypeStruct((NC, 2, T), jnp.float32),
        mesh=_mesh(),
        scratch_types=[
            pltpu.VMEM((CH, 128), jnp.int32),
            pltpu.VMEM((CH, 128), jnp.int32),
            pltpu.VMEM((CH, 128), jnp.float32),
            pltpu.VMEM((CH, 128), jnp.float32),
            pltpu.VMEM_SHARED((T,), jnp.float32),
            pltpu.VMEM_SHARED((T,), jnp.float32),
            pltpu.VMEM_SHARED((T,), jnp.float32),
            pltpu.VMEM_SHARED((T,), jnp.float32),
            pltpu.SemaphoreType.DMA,
        ],
    )(srcp, dstp, p0_t, p1_t, zeros_t)


def _glue1_body(degp_ref, x_ref, dis_ref, s_ref):
    deg = degp_ref[0] + degp_ref[1] + 1.0
    dis = lax.rsqrt(deg)
    dis_ref[...] = dis
    s_ref[...] = dis * x_ref[...]


def _glue1(degp, xpad):
    return pl.pallas_call(
        _glue1_body,
        out_shape=[jax.ShapeDtypeStruct((ROWS, 128), jnp.float32)] * 2,
    )(degp.reshape(NC, ROWS, 128), xpad)


def _glue2_body(accp_ref, s_ref, dis_ref, prm_ref, p0_ref, p1_ref):
    dis = dis_ref[...]
    u = dis * (accp_ref[0] + accp_ref[1] + s_ref[...])
    h0 = u * prm_ref[0] + prm_ref[2]
    h1 = u * prm_ref[1] + prm_ref[3]
    p0_ref[...] = dis * jnp.maximum(h0, 0.0)
    p1_ref[...] = dis * jnp.maximum(h1, 0.0)


def _glue2(accp, s, dis, prm1):
    return pl.pallas_call(
        _glue2_body,
        in_specs=[
            pl.BlockSpec(memory_space=pltpu.MemorySpace.VMEM),
            pl.BlockSpec(memory_space=pltpu.MemorySpace.VMEM),
            pl.BlockSpec(memory_space=pltpu.MemorySpace.VMEM),
            pl.BlockSpec(memory_space=pltpu.MemorySpace.SMEM),
        ],
        out_shape=[jax.ShapeDtypeStruct((ROWS, 128), jnp.float32)] * 2,
    )(accp.reshape(NC, ROWS, 128), s, dis, prm1)


def _glue3_body(a0_ref, a1_ref, p0_ref, p1_ref, dis_ref, prm_ref,
                o0_ref, o1_ref):
    dis = dis_ref[...]
    t0 = dis * (a0_ref[0] + a0_ref[1] + p0_ref[...])
    t1 = dis * (a1_ref[0] + a1_ref[1] + p1_ref[...])
    o0 = t0 * prm_ref[0] + t1 * prm_ref[2] + prm_ref[4]
    o1 = t0 * prm_ref[1] + t1 * prm_ref[3] + prm_ref[5]
    m = jnp.maximum(o0, o1)
    lse = m + jnp.log(jnp.exp(o0 - m) + jnp.exp(o1 - m))
    o0_ref[...] = o0 - lse
    o1_ref[...] = o1 - lse


def _glue3(a0, a1, p0, p1, dis, prm2):
    return pl.pallas_call(
        _glue3_body,
        in_specs=[
            pl.BlockSpec(memory_space=pltpu.MemorySpace.VMEM),
            pl.BlockSpec(memory_space=pltpu.MemorySpace.VMEM),
            pl.BlockSpec(memory_space=pltpu.MemorySpace.VMEM),
            pl.BlockSpec(memory_space=pltpu.MemorySpace.VMEM),
            pl.BlockSpec(memory_space=pltpu.MemorySpace.VMEM),
            pl.BlockSpec(memory_space=pltpu.MemorySpace.SMEM),
        ],
        out_shape=[jax.ShapeDtypeStruct((ROWS, 128), jnp.float32)] * 2,
    )(a0, a1, p0, p1, dis, prm2)


@jax.jit
def kernel(x, edge_index, W1, b1, W2, b2):
    src = edge_index[0].astype(jnp.int32)
    dst = edge_index[1].astype(jnp.int32)
    pad = EPAD - E
    srcp = jnp.pad(src, (0, pad), constant_values=N).reshape(NW, ROWS, 128)
    dstp = jnp.pad(dst, (0, pad), constant_values=N).reshape(NW, ROWS, 128)

    zeros_t = jnp.zeros((T,), jnp.float32)
    xpad = jnp.pad(x[:, 0], (0, T - N)).reshape(ROWS, 128)

    degp = _sc_deg(dstp, zeros_t)
    dis, s = _glue1(degp, xpad)

    s_t = s.reshape(T)
    acc1p = _sc_agg1(srcp, dstp, s_t, zeros_t)

    prm1 = jnp.concatenate([W1[0], b1]).astype(jnp.float32)
    p0, p1 = _glue2(acc1p, s, dis, prm1)

    acc2p = _sc_agg2(srcp, dstp, p0.reshape(T), p1.reshape(T), zeros_t)

    a0 = acc2p[:, 0, :].reshape(NC, ROWS, 128)
    a1 = acc2p[:, 1, :].reshape(NC, ROWS, 128)
    prm2 = jnp.concatenate([W2[0], W2[1], b2]).astype(jnp.float32)
    o0, o1 = _glue3(a0, a1, p0, p1, dis, prm2)

    out = jnp.stack([o0.reshape(T)[:N], o1.reshape(T)[:N]], axis=-1)
    return out


# speedup vs baseline: 197.4101x; 2.4354x over previous
import jax
import jax.numpy as jnp
from jax import lax
from jax.experimental import pallas as pl
from jax.experimental.pallas import tpu as pltpu
from jax.experimental.pallas import tpu_sc as plsc

N = 100_000
E = 3_200_000
NC, NS = 2, 16
NW = NC * NS
ROWS = 784
EPT = ROWS * 128
EPAD = NW * EPT
T = 100_352
CH = 8
NCH = ROWS // CH
TSL = T // NS


def _mesh():
    return plsc.VectorSubcoreMesh(core_axis_name="c", subcore_axis_name="s",
                                  num_cores=NC, num_subcores=NS)


def _make_pass(ncols, gather):
    def body(*refs):
        it = iter(refs)
        src_hbm = next(it) if gather else None
        dst_hbm = next(it)
        tabs_hbm = [next(it) for _ in range(ncols)] if gather else []
        zeros_hbm = next(it)
        out_hbm = next(it)
        sidx_v = next(it) if gather else None
        didx_v = next(it)
        if gather:
            vals_v = [next(it) for _ in range(ncols)]
        else:
            ones_v = next(it)
        tabs_sp = [next(it) for _ in range(ncols)] if gather else []
        accs_sp = [next(it) for _ in range(ncols)]
        semi = next(it)
        semg = next(it)
        sems = next(it)

        cid = lax.axis_index("c")
        sid = lax.axis_index("s")
        wid = cid * NS + sid

        if not gather:
            for c0 in range(0, 128, 16):
                ones_v[0, pl.ds(c0, 16)] = jnp.full((16,), 1.0, jnp.float32)

        @pl.when(sid == 0)
        def _():
            for k in range(ncols):
                if gather:
                    pltpu.sync_copy(tabs_hbm[k], tabs_sp[k])
                pltpu.sync_copy(zeros_hbm, accs_sp[k])
        plsc.subcore_barrier()

        def fire_idx(c, b):
            if gather:
                pltpu.async_copy(src_hbm.at[wid, pl.ds(c * CH, CH)],
                                 sidx_v.at[b], semi)
            pltpu.async_copy(dst_hbm.at[wid, pl.ds(c * CH, CH)],
                             didx_v.at[b], semi)

        def wait_idx(c, b):
            if gather:
                pltpu.make_async_copy(src_hbm.at[wid, pl.ds(c * CH, CH)],
                                      sidx_v.at[b], semi).wait()
            pltpu.make_async_copy(dst_hbm.at[wid, pl.ds(c * CH, CH)],
                                  didx_v.at[b], semi).wait()

        def val_ref(k, b, j):
            return vals_v[k].at[b, j] if gather else ones_v.at[0]

        def drain_scatters(b):
            for k in range(ncols):
                for j in range(CH):
                    pltpu.make_async_copy(val_ref(k, b, j),
                                          accs_sp[k].at[didx_v.at[b, j]],
                                          sems).wait()

        fire_idx(0, 0)

        @pl.loop(0, NCH)
        def _(c):
            b = lax.rem(c, 2)
            wait_idx(c, b)
            if gather:
                for k in range(ncols):
                    for j in range(CH):
                        pltpu.async_copy(tabs_sp[k].at[sidx_v.at[b, j]],
                                         vals_v[k].at[b, j], semg)

            @pl.when(c > 0)
            def _():
                drain_scatters(1 - b)

            @pl.when(c < NCH - 1)
            def _():
                fire_idx(c + 1, 1 - b)

            if gather:
                for k in range(ncols):
                    for j in range(CH):
                        pltpu.make_async_copy(tabs_sp[k].at[sidx_v.at[b, j]],
                                              vals_v[k].at[b, j], semg).wait()
            for k in range(ncols):
                for j in range(CH):
                    pltpu.async_copy(val_ref(k, b, j),
                                     accs_sp[k].at[didx_v.at[b, j]],
                                     sems, add=True)

        drain_scatters((NCH - 1) % 2)
        plsc.subcore_barrier()
        for k in range(ncols):
            pltpu.sync_copy(accs_sp[k].at[pl.ds(sid * TSL, TSL)],
                            out_hbm.at[cid, k, pl.ds(sid * TSL, TSL)])

    scratch = []
    if gather:
        scratch.append(pltpu.VMEM((2, CH, 128), jnp.int32))
    scratch.append(pltpu.VMEM((2, CH, 128), jnp.int32))
    if gather:
        scratch += [pltpu.VMEM((2, CH, 128), jnp.float32)] * ncols
    else:
        scratch.append(pltpu.VMEM((1, 128), jnp.float32))
    if gather:
        scratch += [pltpu.VMEM_SHARED((T,), jnp.float32)] * ncols
    scratch += [pltpu.VMEM_SHARED((T,), jnp.float32)] * ncols
    scratch += [pltpu.SemaphoreType.DMA] * 3

    def call(*inputs):
        return pl.kernel(
            body,
            out_type=jax.ShapeDtypeStruct((NC, ncols, T), jnp.float32),
            mesh=_mesh(),
            scratch_types=scratch,
        )(*inputs)

    return call


_sc_deg = _make_pass(1, gather=False)
_sc_agg1 = _make_pass(1, gather=True)
_sc_agg2 = _make_pass(2, gather=True)


def _glue1_body(degp_ref, x_ref, dis_ref, s_ref):
    deg = degp_ref[0] + degp_ref[1] + 1.0
    dis = lax.rsqrt(deg)
    dis_ref[...] = dis
    s_ref[...] = dis * x_ref[...]


def _glue1(degp, xpad):
    return pl.pallas_call(
        _glue1_body,
        out_shape=[jax.ShapeDtypeStruct((ROWS, 128), jnp.float32)] * 2,
    )(degp.reshape(NC, ROWS, 128), xpad)


def _glue2_body(accp_ref, s_ref, dis_ref, prm_ref, p0_ref, p1_ref):
    dis = dis_ref[...]
    u = dis * (accp_ref[0] + accp_ref[1] + s_ref[...])
    h0 = u * prm_ref[0] + prm_ref[2]
    h1 = u * prm_ref[1] + prm_ref[3]
    p0_ref[...] = dis * jnp.maximum(h0, 0.0)
    p1_ref[...] = dis * jnp.maximum(h1, 0.0)


def _glue2(accp, s, dis, prm1):
    return pl.pallas_call(
        _glue2_body,
        in_specs=[
            pl.BlockSpec(memory_space=pltpu.MemorySpace.VMEM),
            pl.BlockSpec(memory_space=pltpu.MemorySpace.VMEM),
            pl.BlockSpec(memory_space=pltpu.MemorySpace.VMEM),
            pl.BlockSpec(memory_space=pltpu.MemorySpace.SMEM),
        ],
        out_shape=[jax.ShapeDtypeStruct((ROWS, 128), jnp.float32)] * 2,
    )(accp.reshape(NC, ROWS, 128), s, dis, prm1)


def _glue3_body(a0_ref, a1_ref, p0_ref, p1_ref, dis_ref, prm_ref,
                o0_ref, o1_ref):
    dis = dis_ref[...]
    t0 = dis * (a0_ref[0] + a0_ref[1] + p0_ref[...])
    t1 = dis * (a1_ref[0] + a1_ref[1] + p1_ref[...])
    o0 = t0 * prm_ref[0] + t1 * prm_ref[2] + prm_ref[4]
    o1 = t0 * prm_ref[1] + t1 * prm_ref[3] + prm_ref[5]
    m = jnp.maximum(o0, o1)
    lse = m + jnp.log(jnp.exp(o0 - m) + jnp.exp(o1 - m))
    o0_ref[...] = o0 - lse
    o1_ref[...] = o1 - lse


def _glue3(a0, a1, p0, p1, dis, prm2):
    return pl.pallas_call(
        _glue3_body,
        in_specs=[
            pl.BlockSpec(memory_space=pltpu.MemorySpace.VMEM),
            pl.BlockSpec(memory_space=pltpu.MemorySpace.VMEM),
            pl.BlockSpec(memory_space=pltpu.MemorySpace.VMEM),
            pl.BlockSpec(memory_space=pltpu.MemorySpace.VMEM),
            pl.BlockSpec(memory_space=pltpu.MemorySpace.VMEM),
            pl.BlockSpec(memory_space=pltpu.MemorySpace.SMEM),
        ],
        out_shape=[jax.ShapeDtypeStruct((ROWS, 128), jnp.float32)] * 2,
    )(a0, a1, p0, p1, dis, prm2)


@jax.jit
def kernel(x, edge_index, W1, b1, W2, b2):
    src = edge_index[0].astype(jnp.int32)
    dst = edge_index[1].astype(jnp.int32)
    pad = EPAD - E
    srcp = jnp.pad(src, (0, pad), constant_values=N).reshape(NW, ROWS, 128)
    dstp = jnp.pad(dst, (0, pad), constant_values=N).reshape(NW, ROWS, 128)

    zeros_t = jnp.zeros((T,), jnp.float32)
    xpad = jnp.pad(x[:, 0], (0, T - N)).reshape(ROWS, 128)

    degp = _sc_deg(dstp, zeros_t)
    dis, s = _glue1(degp[:, 0, :], xpad)

    acc1p = _sc_agg1(srcp, dstp, s.reshape(T), zeros_t)

    prm1 = jnp.concatenate([W1[0], b1]).astype(jnp.float32)
    p0, p1 = _glue2(acc1p[:, 0, :], s, dis, prm1)

    acc2p = _sc_agg2(srcp, dstp, p0.reshape(T), p1.reshape(T), zeros_t)

    a0 = acc2p[:, 0, :].reshape(NC, ROWS, 128)
    a1 = acc2p[:, 1, :].reshape(NC, ROWS, 128)
    prm2 = jnp.concatenate([W2[0], W2[1], b2]).astype(jnp.float32)
    o0, o1 = _glue3(a0, a1, p0, p1, dis, prm2)

    out = jnp.stack([o0.reshape(T)[:N], o1.reshape(T)[:N]], axis=-1)
    return out
